# speedup vs baseline: 5.4451x; 5.4451x over previous
import functools

import jax
import jax.numpy as jnp
from jax import lax
from jax.experimental import pallas as pl
from jax.experimental.pallas import tpu as pltpu
from jax.experimental.pallas import tpu_sc as plsc

N = 10000
E = 160000
NPAD = 10240
EPAD = 163840
EPT = 10240
W = 128
NWIN = EPT // W
RPT = NPAD // 16


def _sc_aggregate(x_flat, src_p, dst_p, want_deg):
    mesh = plsc.VectorSubcoreMesh(core_axis_name="c", subcore_axis_name="s")

    @functools.partial(
        pl.kernel,
        mesh=mesh,
        out_type=[
            jax.ShapeDtypeStruct((NPAD, 128), jnp.float32),
            jax.ShapeDtypeStruct((NPAD, 128), jnp.float32),
            jax.ShapeDtypeStruct((NPAD,), jnp.float32),
        ],
        scratch_types=[
            pltpu.VMEM((NWIN, W), jnp.int32),
            pltpu.VMEM((NWIN, W), jnp.int32),
            pltpu.VMEM((W, 128), jnp.float32),
            pltpu.VMEM((W,), jnp.float32),
            pltpu.VMEM_SHARED((NPAD, 128), jnp.float32),
            pltpu.VMEM_SHARED((NPAD,), jnp.float32),
        ],
    )
    def body(x_hbm, src_hbm, dst_hbm, agg0_hbm, agg1_hbm, deg_hbm,
             srcw, dstw, rowbuf, ones, acc, dacc):
        c = lax.axis_index("c")
        t = lax.axis_index("s")

        pltpu.sync_copy(src_hbm.at[t], srcw)
        pltpu.sync_copy(dst_hbm.at[t], dstw)

        @pl.when(c == 1)
        def _():
            @pl.loop(0, NWIN)
            def _(i):
                for j in range(W // 16):
                    srcw[i, pl.ds(j * 16, 16)] = srcw[i, pl.ds(j * 16, 16)] + N

        zeros16 = jnp.zeros((16,), jnp.float32)

        @pl.loop(0, W)
        def _(i):
            for j in range(128 // 16):
                rowbuf[i, pl.ds(j * 16, 16)] = zeros16

        for k in range(RPT // W):
            pltpu.sync_copy(rowbuf, acc.at[pl.ds(t * RPT + k * W, W)])
            pltpu.sync_copy(rowbuf.at[0], dacc.at[pl.ds(t * RPT + k * W, W)])
        for j in range(W // 16):
            ones[pl.ds(j * 16, 16)] = jnp.full((16,), 1.0, jnp.float32)

        plsc.subcore_barrier()

        @pl.loop(0, NWIN)
        def _(w):
            pltpu.sync_copy(x_hbm.at[srcw.at[w]], rowbuf)
            pltpu.sync_copy(rowbuf, acc.at[dstw.at[w]], add=True)
            if want_deg:
                @pl.when(c == 0)
                def _():
                    pltpu.sync_copy(ones, dacc.at[dstw.at[w]], add=True)

        plsc.subcore_barrier()

        @pl.when(c == 0)
        def _():
            for k in range(RPT // W):
                sl = pl.ds(t * RPT + k * W, W)
                pltpu.sync_copy(acc.at[sl], agg0_hbm.at[sl])
            if want_deg:
                pltpu.sync_copy(dacc.at[pl.ds(t * RPT, RPT)],
                                deg_hbm.at[pl.ds(t * RPT, RPT)])

        @pl.when(c == 1)
        def _():
            for k in range(RPT // W):
                sl = pl.ds(t * RPT + k * W, W)
                pltpu.sync_copy(acc.at[sl], agg1_hbm.at[sl])

    return body(x_flat, src_p, dst_p)


def _tc_layer(x_st, agg0, agg1, deg, ws_t, wn_t, b, relu, split_out):
    BR = 1000
    grid = (N // BR,)

    def body(x0_ref, x1_ref, a0_ref, a1_ref, deg_ref, ws_ref, wn_ref, b_ref,
             out_ref):
        r = 1.0 / jnp.maximum(deg_ref[...], 1.0)
        a0 = a0_ref[...] * r
        a1 = a1_ref[...] * r
        acc = jnp.dot(x0_ref[0], ws_ref[:128, :],
                      preferred_element_type=jnp.float32)
        acc += jnp.dot(x1_ref[0], ws_ref[128:, :],
                       preferred_element_type=jnp.float32)
        acc += jnp.dot(a0, wn_ref[:128, :], preferred_element_type=jnp.float32)
        acc += jnp.dot(a1, wn_ref[128:, :], preferred_element_type=jnp.float32)
        acc += b_ref[...]
        if relu:
            acc = jnp.maximum(acc, 0.0)
        if split_out:
            out_ref[0] = acc[:, :128]
            out_ref[1] = acc[:, 128:]
        else:
            out_ref[...] = acc

    in_specs = [
        pl.BlockSpec((1, BR, 128), lambda i: (0, i, 0)),
        pl.BlockSpec((1, BR, 128), lambda i: (1, i, 0)),
        pl.BlockSpec((BR, 128), lambda i: (i, 0)),
        pl.BlockSpec((BR, 128), lambda i: (i, 0)),
        pl.BlockSpec((BR, 1), lambda i: (i, 0)),
        pl.BlockSpec((256, 256), lambda i: (0, 0)),
        pl.BlockSpec((256, 256), lambda i: (0, 0)),
        pl.BlockSpec((1, 256), lambda i: (0, 0)),
    ]
    if split_out:
        out_shape = jax.ShapeDtypeStruct((2, N, 128), jnp.float32)
        out_spec = pl.BlockSpec((2, BR, 128), lambda i: (0, i, 0))
    else:
        out_shape = jax.ShapeDtypeStruct((N, 256), jnp.float32)
        out_spec = pl.BlockSpec((BR, 256), lambda i: (i, 0))

    return pl.pallas_call(
        body,
        grid=grid,
        in_specs=in_specs,
        out_specs=out_spec,
        out_shape=out_shape,
    )(x_st, x_st, agg0, agg1, deg, ws_t, wn_t, b)


def kernel(x, edge_index, W_self1, W_neigh1, b1, W_self2, W_neigh2, b2):
    ei = edge_index.astype(jnp.int32)
    npad = EPAD - E
    pad_src = (jnp.arange(npad, dtype=jnp.int32) * 37) % N
    pad_dst = N + (jnp.arange(npad, dtype=jnp.int32) % 64)
    src_p = jnp.concatenate([ei[0], pad_src]).reshape(16, NWIN, W)
    dst_p = jnp.concatenate([ei[1], pad_dst]).reshape(16, NWIN, W)

    x_st = jnp.stack([x[:, :128], x[:, 128:]])

    ws1t = W_self1.T
    wn1t = W_neigh1.T
    ws2t = W_self2.T
    wn2t = W_neigh2.T
    b1r = b1.reshape(1, 256)
    b2r = b2.reshape(1, 256)

    agg0, agg1, deg = _sc_aggregate(x_st.reshape(2 * N, 128), src_p, dst_p,
                                    want_deg=True)
    deg2 = deg.reshape(NPAD, 1)
    h_st = _tc_layer(x_st, agg0[:N], agg1[:N], deg2[:N], ws1t, wn1t, b1r,
                     relu=True, split_out=True)
    agg0b, agg1b, _ = _sc_aggregate(h_st.reshape(2 * N, 128), src_p, dst_p,
                                    want_deg=False)
    out = _tc_layer(h_st, agg0b[:N], agg1b[:N], deg2[:N], ws2t, wn2t, b2r,
                    relu=False, split_out=False)
    return out


# speedup vs baseline: 7.0101x; 1.2874x over previous
import functools

import jax
import jax.numpy as jnp
from jax import lax
from jax.experimental import pallas as pl
from jax.experimental.pallas import tpu as pltpu
from jax.experimental.pallas import tpu_sc as plsc

N = 10000
E = 160000
NPAD = 10240
EPAD = 163840
EPT = 10240
W = 128
NWIN = EPT // W
RPT = NPAD // 16
NB = 2
NI = 2


def _sc_aggregate(x_flat, src2_p, dst_p, want_deg):
    mesh = plsc.VectorSubcoreMesh(core_axis_name="c", subcore_axis_name="s")

    @functools.partial(
        pl.kernel,
        mesh=mesh,
        out_type=[
            jax.ShapeDtypeStruct((NPAD, 128), jnp.float32),
            jax.ShapeDtypeStruct((NPAD, 128), jnp.float32),
            jax.ShapeDtypeStruct((NPAD,), jnp.float32),
        ],
        scratch_types=[
            pltpu.VMEM((NI, W), jnp.int32),
            pltpu.VMEM((NWIN, W), jnp.int32),
            pltpu.VMEM((NB, W, 128), jnp.float32),
            pltpu.VMEM((W,), jnp.float32),
            pltpu.VMEM_SHARED((NPAD, 128), jnp.float32),
            pltpu.VMEM_SHARED((NPAD,), jnp.float32),
        ]
        + [pltpu.SemaphoreType.DMA] * (NB + NB + NI + 2),
    )
    def body(x_hbm, src_hbm, dst_hbm, agg0_hbm, agg1_hbm, deg_hbm,
             iring, dstw, rowbuf, ones, acc, dacc, *sems):
        semg = sems[:NB]
        sems_ = sems[NB:2 * NB]
        semi = sems[2 * NB:2 * NB + NI]
        semd = sems[2 * NB + NI]
        semm = sems[2 * NB + NI + 1]
        c = lax.axis_index("c")
        t = lax.axis_index("s")

        def i_start(w, i):
            pltpu.async_copy(src_hbm.at[c, t, w], iring.at[i], semi[i])

        def i_wait(w, i):
            pltpu.make_async_copy(src_hbm.at[c, t, w], iring.at[i],
                                  semi[i]).wait()

        def g_start(w, b):
            pltpu.async_copy(x_hbm.at[iring.at[b]], rowbuf.at[b], semg[b])

        def g_wait(w, b):
            pltpu.make_async_copy(x_hbm.at[iring.at[b]], rowbuf.at[b],
                                  semg[b]).wait()

        def s_start(w, b):
            pltpu.async_copy(rowbuf.at[b], acc.at[dstw.at[w]], sems_[b],
                             add=True)

        def s_wait(w, b):
            pltpu.make_async_copy(rowbuf.at[b], acc.at[dstw.at[w]],
                                  sems_[b]).wait()

        for w in range(NI):
            i_start(w, w)
        pltpu.sync_copy(dst_hbm.at[t], dstw)

        zeros16 = jnp.zeros((16,), jnp.float32)
        zbuf = rowbuf.at[0]

        @pl.loop(0, W)
        def _(i):
            for j in range(128 // 16):
                zbuf[i, pl.ds(j * 16, 16)] = zeros16

        for k in range(RPT // W):
            pltpu.async_copy(zbuf, acc.at[pl.ds(t * RPT + k * W, W)], semm)
            pltpu.async_copy(zbuf.at[0], dacc.at[pl.ds(t * RPT + k * W, W)],
                             semm)
        for j in range(W // 16):
            ones[pl.ds(j * 16, 16)] = jnp.full((16,), 1.0, jnp.float32)
        for k in range(RPT // W):
            pltpu.make_async_copy(zbuf, acc.at[pl.ds(t * RPT + k * W, W)],
                                  semm).wait()
            pltpu.make_async_copy(zbuf.at[0],
                                  dacc.at[pl.ds(t * RPT + k * W, W)],
                                  semm).wait()

        plsc.subcore_barrier()

        i_wait(0, 0)
        g_start(0, 0)

        @pl.loop(0, NWIN, step=NB)
        def _(w0):
            for b in range(NB):
                w = w0 + b
                g_wait(w, b)
                s_start(w, b)
                if want_deg:
                    @pl.when(c == 0)
                    def _():
                        pltpu.async_copy(ones, dacc.at[dstw.at[w]], semd,
                                         add=True)

                @pl.when(w + 2 < NWIN)
                def _():
                    i_start(w + 2, b)

                @pl.when(w + 1 < NWIN)
                def _():
                    @pl.when(w >= 1)
                    def _():
                        s_wait(w - 1, 1 - b)
                    i_wait(w + 1, 1 - b)
                    g_start(w + 1, 1 - b)

        for w in range(NWIN - NB, NWIN):
            s_wait(w, w % NB)
        if want_deg:
            @pl.when(c == 0)
            def _():
                @pl.loop(0, NWIN)
                def _(w):
                    pltpu.make_async_copy(ones, dacc.at[dstw.at[w]],
                                          semd).wait()

        plsc.subcore_barrier()

        @pl.when(c == 0)
        def _():
            for k in range(RPT // W):
                sl = pl.ds(t * RPT + k * W, W)
                pltpu.async_copy(acc.at[sl], agg0_hbm.at[sl], semm)
            if want_deg:
                pltpu.async_copy(dacc.at[pl.ds(t * RPT, RPT)],
                                 deg_hbm.at[pl.ds(t * RPT, RPT)], semm)
            for k in range(RPT // W):
                sl = pl.ds(t * RPT + k * W, W)
                pltpu.make_async_copy(acc.at[sl], agg0_hbm.at[sl], semm).wait()
            if want_deg:
                pltpu.make_async_copy(dacc.at[pl.ds(t * RPT, RPT)],
                                      deg_hbm.at[pl.ds(t * RPT, RPT)],
                                      semm).wait()

        @pl.when(c == 1)
        def _():
            for k in range(RPT // W):
                sl = pl.ds(t * RPT + k * W, W)
                pltpu.async_copy(acc.at[sl], agg1_hbm.at[sl], semm)
            for k in range(RPT // W):
                sl = pl.ds(t * RPT + k * W, W)
                pltpu.make_async_copy(acc.at[sl], agg1_hbm.at[sl], semm).wait()

    return body(x_flat, src2_p, dst_p)


def _tc_layer(x_st, agg0, agg1, deg, ws_t, wn_t, b, relu, split_out):
    BR = 1000
    grid = (N // BR,)

    def body(x0_ref, x1_ref, a0_ref, a1_ref, deg_ref, ws_ref, wn_ref, b_ref,
             out_ref):
        r = 1.0 / jnp.maximum(deg_ref[...], 1.0)
        a0 = a0_ref[...] * r
        a1 = a1_ref[...] * r
        acc = jnp.dot(x0_ref[0], ws_ref[:128, :],
                      preferred_element_type=jnp.float32)
        acc += jnp.dot(x1_ref[0], ws_ref[128:, :],
                       preferred_element_type=jnp.float32)
        acc += jnp.dot(a0, wn_ref[:128, :], preferred_element_type=jnp.float32)
        acc += jnp.dot(a1, wn_ref[128:, :], preferred_element_type=jnp.float32)
        acc += b_ref[...]
        if relu:
            acc = jnp.maximum(acc, 0.0)
        if split_out:
            out_ref[0] = acc[:, :128]
            out_ref[1] = acc[:, 128:]
        else:
            out_ref[...] = acc

    in_specs = [
        pl.BlockSpec((1, BR, 128), lambda i: (0, i, 0)),
        pl.BlockSpec((1, BR, 128), lambda i: (1, i, 0)),
        pl.BlockSpec((BR, 128), lambda i: (i, 0)),
        pl.BlockSpec((BR, 128), lambda i: (i, 0)),
        pl.BlockSpec((BR, 1), lambda i: (i, 0)),
        pl.BlockSpec((256, 256), lambda i: (0, 0)),
        pl.BlockSpec((256, 256), lambda i: (0, 0)),
        pl.BlockSpec((1, 256), lambda i: (0, 0)),
    ]
    if split_out:
        out_shape = jax.ShapeDtypeStruct((2, N, 128), jnp.float32)
        out_spec = pl.BlockSpec((2, BR, 128), lambda i: (0, i, 0))
    else:
        out_shape = jax.ShapeDtypeStruct((N, 256), jnp.float32)
        out_spec = pl.BlockSpec((BR, 256), lambda i: (i, 0))

    return pl.pallas_call(
        body,
        grid=grid,
        in_specs=in_specs,
        out_specs=out_spec,
        out_shape=out_shape,
    )(x_st, x_st, agg0, agg1, deg, ws_t, wn_t, b)


def kernel(x, edge_index, W_self1, W_neigh1, b1, W_self2, W_neigh2, b2):
    ei = edge_index.astype(jnp.int32)
    npad = EPAD - E
    pad_src = (jnp.arange(npad, dtype=jnp.int32) * 37) % N
    pad_dst = N + (jnp.arange(npad, dtype=jnp.int32) % 64)
    src_p = jnp.concatenate([ei[0], pad_src]).reshape(16, NWIN, W)
    dst_p = jnp.concatenate([ei[1], pad_dst]).reshape(16, NWIN, W)
    src2_p = jnp.stack([src_p, src_p + N])

    x_st = jnp.stack([x[:, :128], x[:, 128:]])

    ws1t = W_self1.T
    wn1t = W_neigh1.T
    ws2t = W_self2.T
    wn2t = W_neigh2.T
    b1r = b1.reshape(1, 256)
    b2r = b2.reshape(1, 256)

    agg0, agg1, deg = _sc_aggregate(x_st.reshape(2 * N, 128), src2_p, dst_p,
                                    want_deg=True)
    deg2 = deg.reshape(NPAD, 1)
    h_st = _tc_layer(x_st, agg0[:N], agg1[:N], deg2[:N], ws1t, wn1t, b1r,
                     relu=True, split_out=True)
    agg0b, agg1b, _ = _sc_aggregate(h_st.reshape(2 * N, 128), src2_p, dst_p,
                                    want_deg=False)
    out = _tc_layer(h_st, agg0b[:N], agg1b[:N], deg2[:N], ws2t, wn2t, b2r,
                    relu=False, split_out=False)
    return out


# speedup vs baseline: 7.0216x; 1.0016x over previous
import functools

import jax
import jax.numpy as jnp
from jax import lax
from jax.experimental import pallas as pl
from jax.experimental.pallas import tpu as pltpu
from jax.experimental.pallas import tpu_sc as plsc

N = 10000
E = 160000
NPAD = 10240
EPAD = 163840
EPT = 10240
W = 128
NWIN = EPT // W
RPT = NPAD // 16
NB = 2
NI = 2


def _sc_aggregate(x_flat, src2_p, dst_p, want_deg):
    mesh = plsc.VectorSubcoreMesh(core_axis_name="c", subcore_axis_name="s")

    @functools.partial(
        pl.kernel,
        mesh=mesh,
        out_type=[
            jax.ShapeDtypeStruct((NPAD, 128), jnp.float32),
            jax.ShapeDtypeStruct((NPAD, 128), jnp.float32),
            jax.ShapeDtypeStruct((NPAD,), jnp.float32),
        ],
        scratch_types=[
            pltpu.VMEM((NI, W), jnp.int32),
            pltpu.VMEM((NWIN, W), jnp.int32),
            pltpu.VMEM((NB, W, 128), jnp.float32),
            pltpu.VMEM((W,), jnp.float32),
            pltpu.VMEM_SHARED((NPAD, 128), jnp.float32),
            pltpu.VMEM_SHARED((NPAD,), jnp.float32),
        ]
        + [pltpu.SemaphoreType.DMA] * (NB + NB + NI + 2),
    )
    def body(x_hbm, src_hbm, dst_hbm, agg0_hbm, agg1_hbm, deg_hbm,
             iring, dstw, rowbuf, ones, acc, dacc, *sems):
        semg = sems[:NB]
        sems_ = sems[NB:2 * NB]
        semi = sems[2 * NB:2 * NB + NI]
        semd = sems[2 * NB + NI]
        semm = sems[2 * NB + NI + 1]
        c = lax.axis_index("c")
        t = lax.axis_index("s")

        def i_start(w, i):
            pltpu.async_copy(src_hbm.at[c, t, w], iring.at[i], semi[i])

        def i_wait(w, i):
            pltpu.make_async_copy(src_hbm.at[c, t, w], iring.at[i],
                                  semi[i]).wait()

        def g_start(w, b):
            pltpu.async_copy(x_hbm.at[iring.at[b]], rowbuf.at[b], semg[b])

        def g_wait(w, b):
            pltpu.make_async_copy(x_hbm.at[iring.at[b]], rowbuf.at[b],
                                  semg[b]).wait()

        def s_start(w, b):
            pltpu.async_copy(rowbuf.at[b], acc.at[dstw.at[w]], sems_[b],
                             add=True)

        def s_wait(w, b):
            pltpu.make_async_copy(rowbuf.at[b], acc.at[dstw.at[w]],
                                  sems_[b]).wait()

        for w in range(NI):
            i_start(w, w)
        pltpu.sync_copy(dst_hbm.at[t], dstw)

        zeros16 = jnp.zeros((16,), jnp.float32)
        zbuf = rowbuf.at[0]

        @pl.loop(0, W)
        def _(i):
            for j in range(128 // 16):
                zbuf[i, pl.ds(j * 16, 16)] = zeros16

        for k in range(RPT // W):
            pltpu.async_copy(zbuf, acc.at[pl.ds(t * RPT + k * W, W)], semm)
            pltpu.async_copy(zbuf.at[0], dacc.at[pl.ds(t * RPT + k * W, W)],
                             semm)
        for j in range(W // 16):
            ones[pl.ds(j * 16, 16)] = jnp.full((16,), 1.0, jnp.float32)
        for k in range(RPT // W):
            pltpu.make_async_copy(zbuf, acc.at[pl.ds(t * RPT + k * W, W)],
                                  semm).wait()
            pltpu.make_async_copy(zbuf.at[0],
                                  dacc.at[pl.ds(t * RPT + k * W, W)],
                                  semm).wait()

        plsc.subcore_barrier()

        i_wait(0, 0)
        g_start(0, 0)

        @pl.loop(0, NWIN, step=NB)
        def _(w0):
            for b in range(NB):
                w = w0 + b
                g_wait(w, b)

                @pl.when(w + 1 < NWIN)
                def _():
                    @pl.when(w >= 1)
                    def _():
                        s_wait(w - 1, 1 - b)
                    i_wait(w + 1, 1 - b)
                    g_start(w + 1, 1 - b)
                s_start(w, b)
                if want_deg:
                    @pl.when(c == 0)
                    def _():
                        pltpu.async_copy(ones, dacc.at[dstw.at[w]], semd,
                                         add=True)

                @pl.when(w + 2 < NWIN)
                def _():
                    i_start(w + 2, b)

        for w in range(NWIN - NB, NWIN):
            s_wait(w, w % NB)
        if want_deg:
            @pl.when(c == 0)
            def _():
                @pl.loop(0, NWIN)
                def _(w):
                    pltpu.make_async_copy(ones, dacc.at[dstw.at[w]],
                                          semd).wait()

        plsc.subcore_barrier()

        @pl.when(c == 0)
        def _():
            for k in range(RPT // W):
                sl = pl.ds(t * RPT + k * W, W)
                pltpu.async_copy(acc.at[sl], agg0_hbm.at[sl], semm)
            if want_deg:
                pltpu.async_copy(dacc.at[pl.ds(t * RPT, RPT)],
                                 deg_hbm.at[pl.ds(t * RPT, RPT)], semm)
            for k in range(RPT // W):
                sl = pl.ds(t * RPT + k * W, W)
                pltpu.make_async_copy(acc.at[sl], agg0_hbm.at[sl], semm).wait()
            if want_deg:
                pltpu.make_async_copy(dacc.at[pl.ds(t * RPT, RPT)],
                                      deg_hbm.at[pl.ds(t * RPT, RPT)],
                                      semm).wait()

        @pl.when(c == 1)
        def _():
            for k in range(RPT // W):
                sl = pl.ds(t * RPT + k * W, W)
                pltpu.async_copy(acc.at[sl], agg1_hbm.at[sl], semm)
            for k in range(RPT // W):
                sl = pl.ds(t * RPT + k * W, W)
                pltpu.make_async_copy(acc.at[sl], agg1_hbm.at[sl], semm).wait()

    return body(x_flat, src2_p, dst_p)


def _tc_layer(x_st, agg0, agg1, deg, ws_t, wn_t, b, relu, split_out):
    BR = 1000
    grid = (N // BR,)

    def body(x0_ref, x1_ref, a0_ref, a1_ref, deg_ref, ws_ref, wn_ref, b_ref,
             out_ref):
        r = 1.0 / jnp.maximum(deg_ref[...], 1.0)
        a0 = a0_ref[...] * r
        a1 = a1_ref[...] * r
        acc = jnp.dot(x0_ref[0], ws_ref[:128, :],
                      preferred_element_type=jnp.float32)
        acc += jnp.dot(x1_ref[0], ws_ref[128:, :],
                       preferred_element_type=jnp.float32)
        acc += jnp.dot(a0, wn_ref[:128, :], preferred_element_type=jnp.float32)
        acc += jnp.dot(a1, wn_ref[128:, :], preferred_element_type=jnp.float32)
        acc += b_ref[...]
        if relu:
            acc = jnp.maximum(acc, 0.0)
        if split_out:
            out_ref[0] = acc[:, :128]
            out_ref[1] = acc[:, 128:]
        else:
            out_ref[...] = acc

    in_specs = [
        pl.BlockSpec((1, BR, 128), lambda i: (0, i, 0)),
        pl.BlockSpec((1, BR, 128), lambda i: (1, i, 0)),
        pl.BlockSpec((BR, 128), lambda i: (i, 0)),
        pl.BlockSpec((BR, 128), lambda i: (i, 0)),
        pl.BlockSpec((BR, 1), lambda i: (i, 0)),
        pl.BlockSpec((256, 256), lambda i: (0, 0)),
        pl.BlockSpec((256, 256), lambda i: (0, 0)),
        pl.BlockSpec((1, 256), lambda i: (0, 0)),
    ]
    if split_out:
        out_shape = jax.ShapeDtypeStruct((2, N, 128), jnp.float32)
        out_spec = pl.BlockSpec((2, BR, 128), lambda i: (0, i, 0))
    else:
        out_shape = jax.ShapeDtypeStruct((N, 256), jnp.float32)
        out_spec = pl.BlockSpec((BR, 256), lambda i: (i, 0))

    return pl.pallas_call(
        body,
        grid=grid,
        in_specs=in_specs,
        out_specs=out_spec,
        out_shape=out_shape,
    )(x_st, x_st, agg0, agg1, deg, ws_t, wn_t, b)


def kernel(x, edge_index, W_self1, W_neigh1, b1, W_self2, W_neigh2, b2):
    ei = edge_index.astype(jnp.int32)
    npad = EPAD - E
    pad_src = (jnp.arange(npad, dtype=jnp.int32) * 37) % N
    pad_dst = N + (jnp.arange(npad, dtype=jnp.int32) % 64)
    src_p = jnp.concatenate([ei[0], pad_src]).reshape(16, NWIN, W)
    dst_p = jnp.concatenate([ei[1], pad_dst]).reshape(16, NWIN, W)
    src2_p = jnp.stack([src_p, src_p + N])

    x_st = jnp.stack([x[:, :128], x[:, 128:]])

    ws1t = W_self1.T
    wn1t = W_neigh1.T
    ws2t = W_self2.T
    wn2t = W_neigh2.T
    b1r = b1.reshape(1, 256)
    b2r = b2.reshape(1, 256)

    agg0, agg1, deg = _sc_aggregate(x_st.reshape(2 * N, 128), src2_p, dst_p,
                                    want_deg=True)
    deg2 = deg.reshape(NPAD, 1)
    h_st = _tc_layer(x_st, agg0[:N], agg1[:N], deg2[:N], ws1t, wn1t, b1r,
                     relu=True, split_out=True)
    agg0b, agg1b, _ = _sc_aggregate(h_st.reshape(2 * N, 128), src2_p, dst_p,
                                    want_deg=False)
    out = _tc_layer(h_st, agg0b[:N], agg1b[:N], deg2[:N], ws2t, wn2t, b2r,
                    relu=False, split_out=False)
    return out


# speedup vs baseline: 7.3548x; 1.0475x over previous
import functools

import jax
import jax.numpy as jnp
import numpy as np
from jax import lax
from jax.experimental import pallas as pl
from jax.experimental.pallas import tpu as pltpu
from jax.experimental.pallas import tpu_sc as plsc

N = 10000
E = 160000
NPAD = 10240
EPAD = 163840
EPT = 10240
W = 128
NWIN = EPT // W
RPT = NPAD // 16
NB = 2
NI = 2


def _sc_aggregate(x_flat, src2_p, dst_p, want_deg):
    mesh = plsc.VectorSubcoreMesh(core_axis_name="c", subcore_axis_name="s")

    @functools.partial(
        pl.kernel,
        mesh=mesh,
        out_type=[
            jax.ShapeDtypeStruct((NPAD, 128), jnp.float32),
            jax.ShapeDtypeStruct((NPAD, 128), jnp.float32),
            jax.ShapeDtypeStruct((NPAD,), jnp.float32),
        ],
        scratch_types=[
            pltpu.VMEM((NI, W), jnp.int32),
            pltpu.VMEM((NWIN, W), jnp.int32),
            pltpu.VMEM((NB, W, 128), jnp.float32),
            pltpu.VMEM((W,), jnp.float32),
            pltpu.VMEM_SHARED((NPAD, 128), jnp.float32),
            pltpu.VMEM_SHARED((NPAD,), jnp.float32),
        ]
        + [pltpu.SemaphoreType.DMA] * (NB + NB + NI + 2),
    )
    def body(x_hbm, src_hbm, dst_hbm, agg0_hbm, agg1_hbm, deg_hbm,
             iring, dstw, rowbuf, ones, acc, dacc, *sems):
        semg = sems[:NB]
        sems_ = sems[NB:2 * NB]
        semi = sems[2 * NB:2 * NB + NI]
        semd = sems[2 * NB + NI]
        semm = sems[2 * NB + NI + 1]
        c = lax.axis_index("c")
        t = lax.axis_index("s")

        def i_start(w, i):
            pltpu.async_copy(src_hbm.at[c, t, w], iring.at[i], semi[i])

        def i_wait(w, i):
            pltpu.make_async_copy(src_hbm.at[c, t, w], iring.at[i],
                                  semi[i]).wait()

        def g_start(w, b):
            pltpu.async_copy(x_hbm.at[iring.at[b]], rowbuf.at[b], semg[b])

        def g_wait(w, b):
            pltpu.make_async_copy(x_hbm.at[iring.at[b]], rowbuf.at[b],
                                  semg[b]).wait()

        def s_start(w, b):
            pltpu.async_copy(rowbuf.at[b], acc.at[dstw.at[w]], sems_[b],
                             add=True)

        def s_wait(w, b):
            pltpu.make_async_copy(rowbuf.at[b], acc.at[dstw.at[w]],
                                  sems_[b]).wait()

        for w in range(NI):
            i_start(w, w)
        pltpu.sync_copy(dst_hbm.at[t], dstw)

        zeros16 = jnp.zeros((16,), jnp.float32)
        zbuf = rowbuf.at[0]

        @pl.loop(0, W)
        def _(i):
            for j in range(128 // 16):
                zbuf[i, pl.ds(j * 16, 16)] = zeros16

        for k in range(RPT // W):
            pltpu.async_copy(zbuf, acc.at[pl.ds(t * RPT + k * W, W)], semm)
            pltpu.async_copy(zbuf.at[0], dacc.at[pl.ds(t * RPT + k * W, W)],
                             semm)
        for j in range(W // 16):
            ones[pl.ds(j * 16, 16)] = jnp.full((16,), 1.0, jnp.float32)
        for k in range(RPT // W):
            pltpu.make_async_copy(zbuf, acc.at[pl.ds(t * RPT + k * W, W)],
                                  semm).wait()
            pltpu.make_async_copy(zbuf.at[0],
                                  dacc.at[pl.ds(t * RPT + k * W, W)],
                                  semm).wait()

        plsc.subcore_barrier()

        i_wait(0, 0)
        g_start(0, 0)

        @pl.loop(0, NWIN, step=NB)
        def _(w0):
            for b in range(NB):
                w = w0 + b
                g_wait(w, b)

                @pl.when(w + 1 < NWIN)
                def _():
                    @pl.when(w >= 1)
                    def _():
                        s_wait(w - 1, 1 - b)
                    i_wait(w + 1, 1 - b)
                    g_start(w + 1, 1 - b)
                s_start(w, b)
                if want_deg:
                    @pl.when(c == 0)
                    def _():
                        pltpu.async_copy(ones, dacc.at[dstw.at[w]], semd,
                                         add=True)

                @pl.when(w + 2 < NWIN)
                def _():
                    i_start(w + 2, b)

        for w in range(NWIN - NB, NWIN):
            s_wait(w, w % NB)
        if want_deg:
            @pl.when(c == 0)
            def _():
                @pl.loop(0, NWIN)
                def _(w):
                    pltpu.make_async_copy(ones, dacc.at[dstw.at[w]],
                                          semd).wait()

        plsc.subcore_barrier()

        @pl.when(c == 0)
        def _():
            for k in range(RPT // W):
                sl = pl.ds(t * RPT + k * W, W)
                pltpu.async_copy(acc.at[sl], agg0_hbm.at[sl], semm)
            if want_deg:
                pltpu.async_copy(dacc.at[pl.ds(t * RPT, RPT)],
                                 deg_hbm.at[pl.ds(t * RPT, RPT)], semm)
            for k in range(RPT // W):
                sl = pl.ds(t * RPT + k * W, W)
                pltpu.make_async_copy(acc.at[sl], agg0_hbm.at[sl], semm).wait()
            if want_deg:
                pltpu.make_async_copy(dacc.at[pl.ds(t * RPT, RPT)],
                                      deg_hbm.at[pl.ds(t * RPT, RPT)],
                                      semm).wait()

        @pl.when(c == 1)
        def _():
            for k in range(RPT // W):
                sl = pl.ds(t * RPT + k * W, W)
                pltpu.async_copy(acc.at[sl], agg1_hbm.at[sl], semm)
            for k in range(RPT // W):
                sl = pl.ds(t * RPT + k * W, W)
                pltpu.make_async_copy(acc.at[sl], agg1_hbm.at[sl], semm).wait()

    return body(x_flat, src2_p, dst_p)


def _tc_layer(x_st, agg0, agg1, deg, ws_t, wn_t, b, relu, split_out):
    BR = 1000
    grid = (N // BR,)

    def body(x0_ref, x1_ref, a0_ref, a1_ref, deg_ref, ws_ref, wn_ref, b_ref,
             out_ref):
        r = 1.0 / jnp.maximum(deg_ref[...], 1.0)
        a0 = a0_ref[...] * r
        a1 = a1_ref[...] * r
        acc = jnp.dot(x0_ref[0], ws_ref[:128, :],
                      preferred_element_type=jnp.float32)
        acc += jnp.dot(x1_ref[0], ws_ref[128:, :],
                       preferred_element_type=jnp.float32)
        acc += jnp.dot(a0, wn_ref[:128, :], preferred_element_type=jnp.float32)
        acc += jnp.dot(a1, wn_ref[128:, :], preferred_element_type=jnp.float32)
        acc += b_ref[...]
        if relu:
            acc = jnp.maximum(acc, 0.0)
        if split_out:
            out_ref[0] = acc[:, :128]
            out_ref[1] = acc[:, 128:]
        else:
            out_ref[...] = acc

    in_specs = [
        pl.BlockSpec((1, BR, 128), lambda i: (0, i, 0)),
        pl.BlockSpec((1, BR, 128), lambda i: (1, i, 0)),
        pl.BlockSpec((BR, 128), lambda i: (i, 0)),
        pl.BlockSpec((BR, 128), lambda i: (i, 0)),
        pl.BlockSpec((BR, 1), lambda i: (i, 0)),
        pl.BlockSpec((256, 256), lambda i: (0, 0)),
        pl.BlockSpec((256, 256), lambda i: (0, 0)),
        pl.BlockSpec((1, 256), lambda i: (0, 0)),
    ]
    if split_out:
        out_shape = jax.ShapeDtypeStruct((2, N, 128), jnp.float32)
        out_spec = pl.BlockSpec((2, BR, 128), lambda i: (0, i, 0))
    else:
        out_shape = jax.ShapeDtypeStruct((N, 256), jnp.float32)
        out_spec = pl.BlockSpec((BR, 256), lambda i: (i, 0))

    return pl.pallas_call(
        body,
        grid=grid,
        in_specs=in_specs,
        out_specs=out_spec,
        out_shape=out_shape,
    )(x_st, x_st, agg0, agg1, deg, ws_t, wn_t, b)


def _tc_self(x_st, ws_t, b):
    BR = 1000
    grid = (N // BR,)

    def body(x0_ref, x1_ref, ws_ref, b_ref, out_ref):
        acc = jnp.dot(x0_ref[0], ws_ref[:128, :],
                      preferred_element_type=jnp.float32)
        acc += jnp.dot(x1_ref[0], ws_ref[128:, :],
                       preferred_element_type=jnp.float32)
        acc += b_ref[...]
        out_ref[0] = acc[:, :128]
        out_ref[1] = acc[:, 128:]

    return pl.pallas_call(
        body,
        grid=grid,
        in_specs=[
            pl.BlockSpec((1, BR, 128), lambda i: (0, i, 0)),
            pl.BlockSpec((1, BR, 128), lambda i: (1, i, 0)),
            pl.BlockSpec((256, 256), lambda i: (0, 0)),
            pl.BlockSpec((1, 256), lambda i: (0, 0)),
        ],
        out_specs=pl.BlockSpec((2, BR, 128), lambda i: (0, i, 0)),
        out_shape=jax.ShapeDtypeStruct((2, N, 128), jnp.float32),
    )(x_st, x_st, ws_t, b)


def _tc_rest(hs_st, agg0, agg1, deg, wn_t, relu, split_out):
    BR = 1000
    grid = (N // BR,)

    def body(h0_ref, h1_ref, a0_ref, a1_ref, deg_ref, wn_ref, out_ref):
        r = 1.0 / jnp.maximum(deg_ref[...], 1.0)
        a0 = a0_ref[...] * r
        a1 = a1_ref[...] * r
        acc = jnp.concatenate([h0_ref[0], h1_ref[0]], axis=1)
        acc += jnp.dot(a0, wn_ref[:128, :], preferred_element_type=jnp.float32)
        acc += jnp.dot(a1, wn_ref[128:, :], preferred_element_type=jnp.float32)
        if relu:
            acc = jnp.maximum(acc, 0.0)
        if split_out:
            out_ref[0] = acc[:, :128]
            out_ref[1] = acc[:, 128:]
        else:
            out_ref[...] = acc

    if split_out:
        out_shape = jax.ShapeDtypeStruct((2, N, 128), jnp.float32)
        out_spec = pl.BlockSpec((2, BR, 128), lambda i: (0, i, 0))
    else:
        out_shape = jax.ShapeDtypeStruct((N, 256), jnp.float32)
        out_spec = pl.BlockSpec((BR, 256), lambda i: (i, 0))
    return pl.pallas_call(
        body,
        grid=grid,
        in_specs=[
            pl.BlockSpec((1, BR, 128), lambda i: (0, i, 0)),
            pl.BlockSpec((1, BR, 128), lambda i: (1, i, 0)),
            pl.BlockSpec((BR, 128), lambda i: (i, 0)),
            pl.BlockSpec((BR, 128), lambda i: (i, 0)),
            pl.BlockSpec((BR, 1), lambda i: (i, 0)),
            pl.BlockSpec((256, 256), lambda i: (0, 0)),
        ],
        out_specs=out_spec,
        out_shape=out_shape,
    )(hs_st, hs_st, agg0, agg1, deg, wn_t)


def kernel(x, edge_index, W_self1, W_neigh1, b1, W_self2, W_neigh2, b2):
    ei = edge_index.astype(jnp.int32)
    npad = EPAD - E
    pad_src = jnp.asarray((np.arange(npad) * 37) % N, jnp.int32)
    pad_dst = jnp.asarray(N + (np.arange(npad) % 64), jnp.int32)
    src_p = jnp.concatenate([ei[0], pad_src]).reshape(16, NWIN, W)
    dst_p = jnp.concatenate([ei[1], pad_dst]).reshape(16, NWIN, W)
    src2_p = jnp.stack([src_p, src_p + N])

    x_st = jnp.stack([x[:, :128], x[:, 128:]])

    ws1t = W_self1.T
    wn1t = W_neigh1.T
    ws2t = W_self2.T
    wn2t = W_neigh2.T
    b1r = b1.reshape(1, 256)
    b2r = b2.reshape(1, 256)

    agg0, agg1, deg = _sc_aggregate(x_st.reshape(2 * N, 128), src2_p, dst_p,
                                    want_deg=True)
    hs1 = _tc_self(x_st, ws1t, b1r)
    deg2 = deg.reshape(NPAD, 1)
    h_st = _tc_rest(hs1, agg0, agg1, deg2, wn1t, relu=True, split_out=True)
    agg0b, agg1b, _ = _sc_aggregate(h_st.reshape(2 * N, 128), src2_p, dst_p,
                                    want_deg=False)
    hs2 = _tc_self(h_st, ws2t, b2r)
    out = _tc_rest(hs2, agg0b, agg1b, deg2, wn2t, relu=False, split_out=False)
    return out


# speedup vs baseline: 7.3594x; 1.0006x over previous
import functools

import jax
import jax.numpy as jnp
import numpy as np
from jax import lax
from jax.experimental import pallas as pl
from jax.experimental.pallas import tpu as pltpu
from jax.experimental.pallas import tpu_sc as plsc

N = 10000
E = 160000
NPAD = 10240
EPAD = 163840
EPT = 10240
W = 128
NWIN = EPT // W
RPT = NPAD // 16
NB = 2
NI = 2


def _sc_aggregate(x_flat, src2_p, dst_p, want_deg):
    mesh = plsc.VectorSubcoreMesh(core_axis_name="c", subcore_axis_name="s")

    @functools.partial(
        pl.kernel,
        mesh=mesh,
        out_type=[
            jax.ShapeDtypeStruct((NPAD, 128), jnp.float32),
            jax.ShapeDtypeStruct((NPAD, 128), jnp.float32),
            jax.ShapeDtypeStruct((NPAD,), jnp.float32),
        ],
        scratch_types=[
            pltpu.VMEM((NI, W), jnp.int32),
            pltpu.VMEM((NWIN, W), jnp.int32),
            pltpu.VMEM((NB, W, 128), jnp.float32),
            pltpu.VMEM((W,), jnp.float32),
            pltpu.VMEM_SHARED((NPAD, 128), jnp.float32),
            pltpu.VMEM_SHARED((NPAD,), jnp.float32),
        ]
        + [pltpu.SemaphoreType.DMA] * (NB + NB + NI + 2),
    )
    def body(x_hbm, src_hbm, dst_hbm, agg0_hbm, agg1_hbm, deg_hbm,
             iring, dstw, rowbuf, ones, acc, dacc, *sems):
        semg = sems[:NB]
        sems_ = sems[NB:2 * NB]
        semi = sems[2 * NB:2 * NB + NI]
        semd = sems[2 * NB + NI]
        semm = sems[2 * NB + NI + 1]
        c = lax.axis_index("c")
        t = lax.axis_index("s")

        def i_start(w, i):
            pltpu.async_copy(src_hbm.at[c, t, w], iring.at[i], semi[i])

        def i_wait(w, i):
            pltpu.make_async_copy(src_hbm.at[c, t, w], iring.at[i],
                                  semi[i]).wait()

        def g_start(w, b):
            pltpu.async_copy(x_hbm.at[iring.at[b]], rowbuf.at[b], semg[b])

        def g_wait(w, b):
            pltpu.make_async_copy(x_hbm.at[iring.at[b]], rowbuf.at[b],
                                  semg[b]).wait()

        def s_start(w, b):
            pltpu.async_copy(rowbuf.at[b], acc.at[dstw.at[w]], sems_[b],
                             add=True)

        def s_wait(w, b):
            pltpu.make_async_copy(rowbuf.at[b], acc.at[dstw.at[w]],
                                  sems_[b]).wait()

        for w in range(NI):
            i_start(w, w)
        pltpu.sync_copy(dst_hbm.at[t], dstw)

        zeros16 = jnp.zeros((16,), jnp.float32)
        zbuf = rowbuf.at[0]

        @pl.loop(0, W)
        def _(i):
            for j in range(128 // 16):
                zbuf[i, pl.ds(j * 16, 16)] = zeros16

        for k in range(RPT // W):
            pltpu.async_copy(zbuf, acc.at[pl.ds(t * RPT + k * W, W)], semm)
            pltpu.async_copy(zbuf.at[0], dacc.at[pl.ds(t * RPT + k * W, W)],
                             semm)
        for j in range(W // 16):
            ones[pl.ds(j * 16, 16)] = jnp.full((16,), 1.0, jnp.float32)
        for k in range(RPT // W):
            pltpu.make_async_copy(zbuf, acc.at[pl.ds(t * RPT + k * W, W)],
                                  semm).wait()
            pltpu.make_async_copy(zbuf.at[0],
                                  dacc.at[pl.ds(t * RPT + k * W, W)],
                                  semm).wait()

        plsc.subcore_barrier()

        i_wait(0, 0)
        g_start(0, 0)

        @pl.loop(0, NWIN, step=NB)
        def _(w0):
            for b in range(NB):
                w = w0 + b
                g_wait(w, b)

                @pl.when(w + 1 < NWIN)
                def _():
                    @pl.when(w >= 1)
                    def _():
                        s_wait(w - 1, 1 - b)
                    i_wait(w + 1, 1 - b)
                    g_start(w + 1, 1 - b)
                s_start(w, b)
                if want_deg:
                    @pl.when(c == 0)
                    def _():
                        pltpu.async_copy(ones, dacc.at[dstw.at[w]], semd,
                                         add=True)

                @pl.when(w + 2 < NWIN)
                def _():
                    i_start(w + 2, b)

        for w in range(NWIN - NB, NWIN):
            s_wait(w, w % NB)
        if want_deg:
            @pl.when(c == 0)
            def _():
                @pl.loop(0, NWIN)
                def _(w):
                    pltpu.make_async_copy(ones, dacc.at[dstw.at[w]],
                                          semd).wait()

        plsc.subcore_barrier()

        @pl.when(c == 0)
        def _():
            for k in range(RPT // W):
                sl = pl.ds(t * RPT + k * W, W)
                pltpu.async_copy(acc.at[sl], agg0_hbm.at[sl], semm)
            if want_deg:
                pltpu.async_copy(dacc.at[pl.ds(t * RPT, RPT)],
                                 deg_hbm.at[pl.ds(t * RPT, RPT)], semm)
            for k in range(RPT // W):
                sl = pl.ds(t * RPT + k * W, W)
                pltpu.make_async_copy(acc.at[sl], agg0_hbm.at[sl], semm).wait()
            if want_deg:
                pltpu.make_async_copy(dacc.at[pl.ds(t * RPT, RPT)],
                                      deg_hbm.at[pl.ds(t * RPT, RPT)],
                                      semm).wait()

        @pl.when(c == 1)
        def _():
            for k in range(RPT // W):
                sl = pl.ds(t * RPT + k * W, W)
                pltpu.async_copy(acc.at[sl], agg1_hbm.at[sl], semm)
            for k in range(RPT // W):
                sl = pl.ds(t * RPT + k * W, W)
                pltpu.make_async_copy(acc.at[sl], agg1_hbm.at[sl], semm).wait()

    return body(x_flat, src2_p, dst_p)


def _tc_self(x_st, ws_t, b):
    BR = 1000
    grid = (N // BR,)

    def body(x0_ref, x1_ref, ws_ref, b_ref, out_ref):
        acc = jnp.dot(x0_ref[0], ws_ref[:128, :],
                      preferred_element_type=jnp.float32)
        acc += jnp.dot(x1_ref[0], ws_ref[128:, :],
                       preferred_element_type=jnp.float32)
        acc += b_ref[...]
        out_ref[0] = acc[:, :128]
        out_ref[1] = acc[:, 128:]

    return pl.pallas_call(
        body,
        grid=grid,
        in_specs=[
            pl.BlockSpec((1, BR, 128), lambda i: (0, i, 0)),
            pl.BlockSpec((1, BR, 128), lambda i: (1, i, 0)),
            pl.BlockSpec((256, 256), lambda i: (0, 0)),
            pl.BlockSpec((1, 256), lambda i: (0, 0)),
        ],
        out_specs=pl.BlockSpec((2, BR, 128), lambda i: (0, i, 0)),
        out_shape=jax.ShapeDtypeStruct((2, N, 128), jnp.float32),
    )(x_st, x_st, ws_t, b)


def _tc_rest(hs_st, agg0, agg1, deg, wn_t, relu, split_out):
    BR = 1000
    grid = (N // BR,)

    def body(h0_ref, h1_ref, a0_ref, a1_ref, deg_ref, wn_ref, out_ref):
        r = 1.0 / jnp.maximum(deg_ref[...], 1.0)
        a0 = a0_ref[...] * r
        a1 = a1_ref[...] * r
        acc = jnp.concatenate([h0_ref[0], h1_ref[0]], axis=1)
        acc += jnp.dot(a0, wn_ref[:128, :], preferred_element_type=jnp.float32)
        acc += jnp.dot(a1, wn_ref[128:, :], preferred_element_type=jnp.float32)
        if relu:
            acc = jnp.maximum(acc, 0.0)
        if split_out:
            out_ref[0] = acc[:, :128]
            out_ref[1] = acc[:, 128:]
        else:
            out_ref[...] = acc

    if split_out:
        out_shape = jax.ShapeDtypeStruct((2, N, 128), jnp.float32)
        out_spec = pl.BlockSpec((2, BR, 128), lambda i: (0, i, 0))
    else:
        out_shape = jax.ShapeDtypeStruct((N, 256), jnp.float32)
        out_spec = pl.BlockSpec((BR, 256), lambda i: (i, 0))
    return pl.pallas_call(
        body,
        grid=grid,
        in_specs=[
            pl.BlockSpec((1, BR, 128), lambda i: (0, i, 0)),
            pl.BlockSpec((1, BR, 128), lambda i: (1, i, 0)),
            pl.BlockSpec((BR, 128), lambda i: (i, 0)),
            pl.BlockSpec((BR, 128), lambda i: (i, 0)),
            pl.BlockSpec((BR, 1), lambda i: (i, 0)),
            pl.BlockSpec((256, 256), lambda i: (0, 0)),
        ],
        out_specs=out_spec,
        out_shape=out_shape,
    )(hs_st, hs_st, agg0, agg1, deg, wn_t)


def kernel(x, edge_index, W_self1, W_neigh1, b1, W_self2, W_neigh2, b2):
    ei = edge_index.astype(jnp.int32)
    npad = EPAD - E
    pad_src = jnp.asarray((np.arange(npad) * 37) % N, jnp.int32)
    pad_dst = jnp.asarray(N + (np.arange(npad) % 64), jnp.int32)
    src_p = jnp.concatenate([ei[0], pad_src]).reshape(16, NWIN, W)
    dst_p = jnp.concatenate([ei[1], pad_dst]).reshape(16, NWIN, W)
    src2_p = jnp.stack([src_p, src_p + N])

    x_st = jnp.stack([x[:, :128], x[:, 128:]])

    ws1t = W_self1.T
    wn1t = W_neigh1.T
    ws2t = W_self2.T
    wn2t = W_neigh2.T
    b1r = b1.reshape(1, 256)
    b2r = b2.reshape(1, 256)

    agg0, agg1, deg = _sc_aggregate(x_st.reshape(2 * N, 128), src2_p, dst_p,
                                    want_deg=True)
    hs1 = _tc_self(x_st, ws1t, b1r)
    deg2 = deg.reshape(NPAD, 1)
    h_st = _tc_rest(hs1, agg0, agg1, deg2, wn1t, relu=True, split_out=True)
    agg0b, agg1b, _ = _sc_aggregate(h_st.reshape(2 * N, 128), src2_p, dst_p,
                                    want_deg=False)
    hs2 = _tc_self(h_st, ws2t, b2r)
    out = _tc_rest(hs2, agg0b, agg1b, deg2, wn2t, relu=False, split_out=False)
    return out
